# Initial kernel scaffold; baseline (speedup 1.0000x reference)
import jax
import jax.numpy as jnp
from jax.experimental import pallas as pl


def kernel(node_x, node_features, edge_index, batch, W1, b1, W2, b2, p, Wa, ba, Wn, bn, Wq, bq, Wkv, bkv, g_ln, b_ln, Wt1, as1, ad1, We1, ae1, bias1, Wt2, as2, ad2, We2, ae2, bias2, A1, c1, A2, c2, A3, c3):
    raise NotImplementedError("write your pallas kernel here")


# speedup vs baseline: 21.0128x; 21.0128x over previous
import jax
import jax.numpy as jnp
from jax.experimental import pallas as pl
from jax.experimental.pallas import tpu as pltpu

G = 50
NG = 200
N = G * NG
H = 128
CD = 16
K = 50

_DEF = jax.lax.Precision.DEFAULT
_HI = jax.lax.Precision.HIGHEST


def _dot(a, b, prec=_DEF):
    return jax.lax.dot_general(a, b, (((1,), (0,)), ((), ())),
                               precision=prec,
                               preferred_element_type=jnp.float32)


def _dot_rt(a, b, prec=_DEF):
    return jax.lax.dot_general(a, b, (((1,), (1,)), ((), ())),
                               precision=prec,
                               preferred_element_type=jnp.float32)


def _eye(n):
    return jnp.where(jax.lax.broadcasted_iota(jnp.int32, (n, n), 0) ==
                     jax.lax.broadcasted_iota(jnp.int32, (n, n), 1),
                     1.0, 0.0).astype(jnp.float32)


def _graph_kernel(nf_ref, p_col_ref, pn_ref,
                  W1_ref, b1_ref, W2_ref, b2_ref,
                  Wa_ref, ba_ref, Wn_ref, bn_ref,
                  Wq_ref, bq_ref,
                  Wk_nf_ref, Wk_ef_ref, bk_ref,
                  Wv_nf_ref, Wv_ef_ref, bv_ref,
                  gln_ref, bln_ref,
                  Wt1_ref, as1_ref, ad1_ref, We1_ref, ae1_ref, bias1_ref,
                  Wt2_ref, as2_ref, ad2_ref, We2_ref, ae2_ref, bias2_ref,
                  A1nf_ref, A1ax_ref, A1ef_ref, c1_ref,
                  A2_ref, c2_ref, A3_ref, c3_ref,
                  out_ref, akl_ref, nkl_ref):
    f32 = jnp.float32
    nf = nf_ref[0]

    h1 = jax.nn.relu(_dot(nf, W1_ref[...]) + b1_ref[...])
    sv = jax.nn.relu(_dot(h1, W2_ref[...]) + b2_ref[...])
    s_col = jnp.tanh(_dot(sv, p_col_ref[...]) / pn_ref[...])
    s_row = jax.lax.dot_general(s_col, _eye(NG), (((0,), (0,)), ((), ())),
                                precision=_HI,
                                preferred_element_type=f32)

    ii = jax.lax.broadcasted_iota(jnp.int32, (NG, NG), 0)
    jj = jax.lax.broadcasted_iota(jnp.int32, (NG, NG), 1)
    beats_c = ((s_row > s_col) | ((s_row == s_col) & (jj < ii))) & (jj != ii)
    beats = jnp.where(beats_c, 1.0, 0.0).astype(f32)
    ones_row = jnp.ones((1, NG), f32)
    cnt_row = _dot_rt(ones_row, beats, _HI)
    p_iota = jax.lax.broadcasted_iota(jnp.int32, (K, NG), 0).astype(f32)
    sel = jnp.where(p_iota == cnt_row, 1.0, 0.0).astype(f32)

    weight = _dot(sel, s_col, _HI)
    af = _dot(sel, sv, _HI) * weight

    ap = _dot(af, Wa_ref[...]) + ba_ref[...]
    npj = _dot(nf, Wn_ref[...]) + bn_ref[...]

    mu_a = jnp.mean(ap, axis=0, keepdims=True)
    da = ap - mu_a
    var_a = jnp.sum(da * da, axis=0, keepdims=True) / (K - 1)
    akl_ref[0] = 0.5 * jnp.sum(var_a + mu_a * mu_a - 1.0 - jnp.log(var_a),
                               axis=1, keepdims=True)

    mu_n = jnp.mean(npj, axis=0, keepdims=True)
    dn = npj - mu_n
    var_n = jnp.sum(dn * dn, axis=0, keepdims=True) / (NG - 1)
    nkl_ref[0] = 0.5 * jnp.sum(var_n + mu_n * mu_n - 1.0 - jnp.log(var_n),
                               axis=1, keepdims=True)

    diff = npj[:, None, :] - ap[None, :, :]
    dist = jnp.sqrt(jnp.sum(diff * diff, axis=-1))
    dmin = jnp.min(dist, axis=1, keepdims=True)
    kidx = jax.lax.broadcasted_iota(jnp.int32, (NG, K), 1).astype(f32)
    argm = jnp.min(jnp.where(dist <= dmin, kidx, float(K)),
                   axis=1, keepdims=True)
    oh = jnp.where(kidx == argm, 1.0, 0.0).astype(f32)
    ohT = _dot_rt(_eye(K), oh, _HI)

    dscore = 1.0 / jnp.sum(jnp.exp(dmin - dist), axis=1, keepdims=True)
    ef = (npj - _dot(oh, ap, _HI)) * dscore

    aq = _dot(af, Wq_ref[...]) + bq_ref[...]
    kk = _dot(nf, Wk_nf_ref[...]) + _dot(ef, Wk_ef_ref[...]) + bk_ref[...]
    vv = _dot(nf, Wv_nf_ref[...]) + _dot(ef, Wv_ef_ref[...]) + bv_ref[...]
    attn = jnp.sum(_dot(oh, aq, _HI) * kk, axis=1, keepdims=True)

    masked = jnp.where(oh > 0.0, attn, -1e30)
    m_row = jnp.max(masked, axis=0, keepdims=True)
    ex = jnp.exp(attn - jnp.sum(oh * m_row, axis=1, keepdims=True))
    den_row = jnp.sum(oh * ex, axis=0, keepdims=True)
    alpha = ex / (jnp.sum(oh * den_row, axis=1, keepdims=True) + 1e-16)
    upd = _dot(ohT, alpha * vv, _HI)

    gln = gln_ref[...]
    bln = bln_ref[...]

    afu = af + upd
    mu = jnp.mean(afu, axis=1, keepdims=True)
    d = afu - mu
    var = jnp.mean(d * d, axis=1, keepdims=True)
    af2 = d * jax.lax.rsqrt(var + 1e-5) * gln + bln

    def gat(x, Wt_ref, as_ref, ad_ref, We_ref, ae_ref, b_ref):
        xt = _dot(x, Wt_ref[...])
        ddot = _dot(xt, ad_ref[...])
        sdot = _dot_rt(as_ref[...], xt)
        we = _dot(We_ref[...], ae_ref[...], _HI)
        ce_col = _dot(ap, we, _HI)
        ce_row = jax.lax.dot_general(ce_col, _eye(K), (((0,), (0,)), ((), ())),
                                     precision=_HI,
                                     preferred_element_type=jnp.float32)
        lg = (ddot - ce_col) + (sdot + ce_row)
        lg = jnp.where(lg >= 0.0, lg, 0.2 * lg)
        rmax = jnp.max(lg, axis=1, keepdims=True)
        e = jnp.exp(lg - rmax)
        a = e / (jnp.sum(e, axis=1, keepdims=True) + 1e-16)
        return _dot(a, xt) + b_ref[...]

    h = jax.nn.relu(gat(af2, Wt1_ref, as1_ref, ad1_ref, We1_ref,
                        ae1_ref, bias1_ref))
    af3 = gat(h, Wt2_ref, as2_ref, ad2_ref, We2_ref,
              ae2_ref, bias2_ref)

    ax = _dot(oh, af3, _HI)
    m1 = jax.nn.relu(_dot(nf, A1nf_ref[...]) + _dot(ax, A1ax_ref[...])
                     + _dot(-ef, A1ef_ref[...]) + c1_ref[...])
    m2 = jax.nn.relu(_dot(m1, A2_ref[...]) + c2_ref[...])
    x = nf + _dot(m2, A3_ref[...]) + c3_ref[...]
    mu = jnp.mean(x, axis=1, keepdims=True)
    d = x - mu
    var = jnp.mean(d * d, axis=1, keepdims=True)
    out_ref[0] = d * jax.lax.rsqrt(var + 1e-5) * gln + bln


def _full(shape):
    nd = len(shape)
    return pl.BlockSpec(shape, lambda g, _nd=nd: (0,) * _nd)


@jax.jit
def kernel(node_x, node_features, edge_index, batch, W1, b1, W2, b2, p,
           Wa, ba, Wn, bn, Wq, bq, Wkv, bkv, g_ln, b_ln,
           Wt1, as1, ad1, We1, ae1, bias1, Wt2, as2, ad2, We2, ae2, bias2,
           A1, c1, A2, c2, A3, c3):
    f32 = jnp.float32
    nf3 = node_features.reshape(G, NG, H)
    row = lambda v: v.reshape(1, -1).astype(f32)
    col = lambda v: v.reshape(-1, 1).astype(f32)
    pn = jnp.linalg.norm(p).reshape(1, 1)

    operands = [
        nf3, col(p), pn,
        W1, row(b1), W2, row(b2),
        Wa, row(ba), Wn, row(bn),
        Wq, row(bq),
        Wkv[:H, :H], Wkv[H:, :H], row(bkv[:H]),
        Wkv[:H, H:], Wkv[H:, H:], row(bkv[H:]),
        row(g_ln), row(b_ln),
        Wt1, row(as1), col(ad1), We1, col(ae1), row(bias1),
        Wt2, row(as2), col(ad2), We2, col(ae2), row(bias2),
        A1[:H], A1[H:2 * H], A1[2 * H:], row(c1),
        A2, row(c2), A3, row(c3),
    ]
    in_specs = [pl.BlockSpec((1, NG, H), lambda g: (g, 0, 0))]
    in_specs += [_full(op.shape) for op in operands[1:]]

    out_shapes = (
        jax.ShapeDtypeStruct((G, NG, H), f32),
        jax.ShapeDtypeStruct((G, 1, 1), f32),
        jax.ShapeDtypeStruct((G, 1, 1), f32),
    )
    out_specs = (
        pl.BlockSpec((1, NG, H), lambda g: (g, 0, 0)),
        pl.BlockSpec((1, 1, 1), lambda g: (g, 0, 0)),
        pl.BlockSpec((1, 1, 1), lambda g: (g, 0, 0)),
    )

    node_out, akl, nkl = pl.pallas_call(
        _graph_kernel,
        grid=(G,),
        in_specs=in_specs,
        out_specs=out_specs,
        out_shape=out_shapes,
        compiler_params=pltpu.CompilerParams(
            dimension_semantics=("arbitrary",)),
    )(*operands)

    return node_out.reshape(N, H), akl.reshape(G), nkl.reshape(G)


# speedup vs baseline: 44.0070x; 2.0943x over previous
import jax
import jax.numpy as jnp
from jax.experimental import pallas as pl
from jax.experimental.pallas import tpu as pltpu

G = 50
NG = 200
N = G * NG
H = 128
CD = 16
K = 50

B = 5
NB = B * NG
KB = B * K

_DEF = jax.lax.Precision.DEFAULT
_HI = jax.lax.Precision.HIGHEST
_NEG = -1e30


def _dot(a, b, prec=_DEF):
    return jax.lax.dot_general(a, b, (((1,), (0,)), ((), ())),
                               precision=prec,
                               preferred_element_type=jnp.float32)


def _dot_rt(a, b, prec=_DEF):
    return jax.lax.dot_general(a, b, (((1,), (1,)), ((), ())),
                               precision=prec,
                               preferred_element_type=jnp.float32)


def _dot_lt(a, b, prec=_HI):
    return jax.lax.dot_general(a, b, (((0,), (0,)), ((), ())),
                               precision=prec,
                               preferred_element_type=jnp.float32)


def _eye(n):
    return jnp.where(jax.lax.broadcasted_iota(jnp.int32, (n, n), 0) ==
                     jax.lax.broadcasted_iota(jnp.int32, (n, n), 1),
                     1.0, 0.0).astype(jnp.float32)


def _iota_col(n, div=1):
    return jax.lax.broadcasted_iota(jnp.int32, (n, 1), 0) // div


def _iota_row(n, div=1):
    return jax.lax.broadcasted_iota(jnp.int32, (1, n), 1) // div


def _graph_kernel(nf_ref, p_col_ref, pn_ref,
                  W1_ref, b1_ref, W2_ref, b2_ref,
                  Wa_ref, ba_ref, Wn_ref, bn_ref,
                  Wq_ref, bq_ref,
                  Wk_nf_ref, Wk_ef_ref, bk_ref,
                  Wv_nf_ref, Wv_ef_ref, bv_ref,
                  gln_ref, bln_ref,
                  Wt1_ref, as1_ref, ad1_ref, We1_ref, ae1_ref, bias1_ref,
                  Wt2_ref, as2_ref, ad2_ref, We2_ref, ae2_ref, bias2_ref,
                  A1nf_ref, A1ax_ref, A1ef_ref, c1_ref,
                  A2_ref, c2_ref, A3_ref, c3_ref,
                  out_ref, akl_ref, nkl_ref):
    f32 = jnp.float32
    nf = nf_ref[0]

    h1 = jax.nn.relu(_dot(nf, W1_ref[...]) + b1_ref[...])
    sv = jax.nn.relu(_dot(h1, W2_ref[...]) + b2_ref[...])
    s_col = jnp.tanh(_dot(sv, p_col_ref[...]) / pn_ref[...])
    s_row = _dot_lt(s_col, _eye(NB))

    ii = jax.lax.broadcasted_iota(jnp.int32, (NB, NB), 0)
    jj = jax.lax.broadcasted_iota(jnp.int32, (NB, NB), 1)
    same_g = _iota_col(NB, NG) == _iota_row(NB, NG)
    beats_c = (((s_row > s_col) | ((s_row == s_col) & (jj < ii)))
               & (jj != ii) & same_g)
    beats = jnp.where(beats_c, 1.0, 0.0).astype(f32)
    ones_row = jnp.ones((1, NB), f32)
    cnt_row = _dot_rt(ones_row, beats, _HI)
    slot_row = cnt_row + (_iota_row(NB, NG) * K).astype(f32)
    p_glob = jax.lax.broadcasted_iota(jnp.int32, (KB, NB), 0).astype(f32)
    sel = jnp.where((p_glob == slot_row) & (cnt_row < float(K)),
                    1.0, 0.0).astype(f32)

    weight = _dot(sel, s_col, _HI)
    af = _dot(sel, sv, _HI) * weight

    ap = _dot(af, Wa_ref[...]) + ba_ref[...]
    npj = _dot(nf, Wn_ref[...]) + bn_ref[...]

    iga = jnp.where(_iota_col(B) == _iota_row(KB, K), 1.0, 0.0).astype(f32)
    mu_a = _dot(iga, ap, _HI) / K
    sq_a = _dot(iga, ap * ap, _HI)
    var_a = (sq_a - K * mu_a * mu_a) / (K - 1)
    akl_ref[0] = 0.5 * jnp.sum(var_a + mu_a * mu_a - 1.0 - jnp.log(var_a),
                               axis=1, keepdims=True)

    ign = jnp.where(_iota_col(B) == _iota_row(NB, NG), 1.0, 0.0).astype(f32)
    mu_n = _dot(ign, npj, _HI) / NG
    sq_n = _dot(ign, npj * npj, _HI)
    var_n = (sq_n - NG * mu_n * mu_n) / (NG - 1)
    nkl_ref[0] = 0.5 * jnp.sum(var_n + mu_n * mu_n - 1.0 - jnp.log(var_n),
                               axis=1, keepdims=True)

    apT = _dot_lt(ap, _eye(KB))
    d2 = jnp.zeros((NB, KB), f32)
    for c in range(CD):
        dc = npj[:, c:c + 1] - apT[c:c + 1, :]
        d2 = d2 + dc * dc
    same_a = _iota_col(NB, NG) == _iota_row(KB, K)
    dist = jnp.where(same_a, jnp.sqrt(d2), 1e30)
    dmin = jnp.min(dist, axis=1, keepdims=True)
    kidx = jax.lax.broadcasted_iota(jnp.int32, (NB, KB), 1).astype(f32)
    argm = jnp.min(jnp.where(dist <= dmin, kidx, float(KB)),
                   axis=1, keepdims=True)
    oh = jnp.where(kidx == argm, 1.0, 0.0).astype(f32)
    ohT = _dot_rt(_eye(KB), oh, _HI)

    dscore = 1.0 / jnp.sum(jnp.exp(dmin - dist), axis=1, keepdims=True)
    ef = (npj - _dot(oh, ap, _HI)) * dscore

    aq = _dot(af, Wq_ref[...]) + bq_ref[...]
    kk = _dot(nf, Wk_nf_ref[...]) + _dot(ef, Wk_ef_ref[...]) + bk_ref[...]
    vv = _dot(nf, Wv_nf_ref[...]) + _dot(ef, Wv_ef_ref[...]) + bv_ref[...]
    attn = jnp.sum(_dot(oh, aq, _HI) * kk, axis=1, keepdims=True)

    masked = jnp.where(oh > 0.0, attn, _NEG)
    m_row = jnp.max(masked, axis=0, keepdims=True)
    ex = jnp.exp(attn - jnp.sum(oh * m_row, axis=1, keepdims=True))
    den_row = jnp.sum(oh * ex, axis=0, keepdims=True)
    alpha = ex / (jnp.sum(oh * den_row, axis=1, keepdims=True) + 1e-16)
    upd = _dot(ohT, alpha * vv, _HI)

    gln = gln_ref[...]
    bln = bln_ref[...]

    afu = af + upd
    mu = jnp.mean(afu, axis=1, keepdims=True)
    d = afu - mu
    var = jnp.mean(d * d, axis=1, keepdims=True)
    af2 = d * jax.lax.rsqrt(var + 1e-5) * gln + bln

    same_aa = _iota_col(KB, K) == _iota_row(KB, K)

    def gat(x, Wt_ref, as_ref, ad_ref, We_ref, ae_ref, b_ref):
        xt = _dot(x, Wt_ref[...])
        ddot = _dot(xt, ad_ref[...])
        sdot = _dot_rt(as_ref[...], xt)
        we = _dot(We_ref[...], ae_ref[...], _HI)
        ce_col = _dot(ap, we, _HI)
        ce_row = _dot_lt(ce_col, _eye(KB))
        lg = (ddot - ce_col) + (sdot + ce_row)
        lg = jnp.where(lg >= 0.0, lg, 0.2 * lg)
        lg = jnp.where(same_aa, lg, _NEG)
        rmax = jnp.max(lg, axis=1, keepdims=True)
        e = jnp.exp(lg - rmax)
        a = e / (jnp.sum(e, axis=1, keepdims=True) + 1e-16)
        return _dot(a, xt) + b_ref[...]

    h = jax.nn.relu(gat(af2, Wt1_ref, as1_ref, ad1_ref, We1_ref,
                        ae1_ref, bias1_ref))
    af3 = gat(h, Wt2_ref, as2_ref, ad2_ref, We2_ref,
              ae2_ref, bias2_ref)

    ax = _dot(oh, af3, _HI)
    m1 = jax.nn.relu(_dot(nf, A1nf_ref[...]) + _dot(ax, A1ax_ref[...])
                     + _dot(-ef, A1ef_ref[...]) + c1_ref[...])
    m2 = jax.nn.relu(_dot(m1, A2_ref[...]) + c2_ref[...])
    x = nf + _dot(m2, A3_ref[...]) + c3_ref[...]
    mu = jnp.mean(x, axis=1, keepdims=True)
    d = x - mu
    var = jnp.mean(d * d, axis=1, keepdims=True)
    out_ref[0] = d * jax.lax.rsqrt(var + 1e-5) * gln + bln


def _full(shape):
    nd = len(shape)
    return pl.BlockSpec(shape, lambda g, _nd=nd: (0,) * _nd)


@jax.jit
def kernel(node_x, node_features, edge_index, batch, W1, b1, W2, b2, p,
           Wa, ba, Wn, bn, Wq, bq, Wkv, bkv, g_ln, b_ln,
           Wt1, as1, ad1, We1, ae1, bias1, Wt2, as2, ad2, We2, ae2, bias2,
           A1, c1, A2, c2, A3, c3):
    f32 = jnp.float32
    nf3 = node_features.reshape(G // B, NB, H)
    row = lambda v: v.reshape(1, -1).astype(f32)
    col = lambda v: v.reshape(-1, 1).astype(f32)
    pn = jnp.linalg.norm(p).reshape(1, 1)

    operands = [
        nf3, col(p), pn,
        W1, row(b1), W2, row(b2),
        Wa, row(ba), Wn, row(bn),
        Wq, row(bq),
        Wkv[:H, :H], Wkv[H:, :H], row(bkv[:H]),
        Wkv[:H, H:], Wkv[H:, H:], row(bkv[H:]),
        row(g_ln), row(b_ln),
        Wt1, row(as1), col(ad1), We1, col(ae1), row(bias1),
        Wt2, row(as2), col(ad2), We2, col(ae2), row(bias2),
        A1[:H], A1[H:2 * H], A1[2 * H:], row(c1),
        A2, row(c2), A3, row(c3),
    ]
    in_specs = [pl.BlockSpec((1, NB, H), lambda g: (g, 0, 0))]
    in_specs += [_full(op.shape) for op in operands[1:]]

    out_shapes = (
        jax.ShapeDtypeStruct((G // B, NB, H), f32),
        jax.ShapeDtypeStruct((G // B, B, 1), f32),
        jax.ShapeDtypeStruct((G // B, B, 1), f32),
    )
    out_specs = (
        pl.BlockSpec((1, NB, H), lambda g: (g, 0, 0)),
        pl.BlockSpec((1, B, 1), lambda g: (g, 0, 0)),
        pl.BlockSpec((1, B, 1), lambda g: (g, 0, 0)),
    )

    node_out, akl, nkl = pl.pallas_call(
        _graph_kernel,
        grid=(G // B,),
        in_specs=in_specs,
        out_specs=out_specs,
        out_shape=out_shapes,
        compiler_params=pltpu.CompilerParams(
            dimension_semantics=("arbitrary",)),
    )(*operands)

    return node_out.reshape(N, H), akl.reshape(G), nkl.reshape(G)


# speedup vs baseline: 53.5611x; 1.2171x over previous
import jax
import jax.numpy as jnp
from jax.experimental import pallas as pl
from jax.experimental.pallas import tpu as pltpu

G = 50
NG = 200
N = G * NG
H = 128
CD = 16
K = 50

B = 5
NB = B * NG
KB = B * K

_DEF = jax.lax.Precision.DEFAULT
_HI = jax.lax.Precision.HIGHEST
_NEG = -1e30


def _dot(a, b, prec=_DEF):
    return jax.lax.dot_general(a, b, (((1,), (0,)), ((), ())),
                               precision=prec,
                               preferred_element_type=jnp.float32)


def _dot_rt(a, b, prec=_DEF):
    return jax.lax.dot_general(a, b, (((1,), (1,)), ((), ())),
                               precision=prec,
                               preferred_element_type=jnp.float32)


def _dot_lt(a, b, prec=_HI):
    return jax.lax.dot_general(a, b, (((0,), (0,)), ((), ())),
                               precision=prec,
                               preferred_element_type=jnp.float32)


def _eye(n):
    return jnp.where(jax.lax.broadcasted_iota(jnp.int32, (n, n), 0) ==
                     jax.lax.broadcasted_iota(jnp.int32, (n, n), 1),
                     1.0, 0.0).astype(jnp.float32)


def _iota_col(n, div=1):
    return jax.lax.broadcasted_iota(jnp.int32, (n, 1), 0) // div


def _iota_row(n, div=1):
    return jax.lax.broadcasted_iota(jnp.int32, (1, n), 1) // div


def _graph_kernel(nf_ref, p_col_ref, pn_ref,
                  W1_ref, b1_ref, W2_ref, b2_ref,
                  Wa_ref, ba_ref, Wn_ref, bn_ref,
                  Wq_ref, bq_ref,
                  Wk_nf_ref, Wk_ef_ref, bk_ref,
                  Wv_nf_ref, Wv_ef_ref, bv_ref,
                  gln_ref, bln_ref,
                  Wt1_ref, as1_ref, ad1_ref, We1_ref, ae1_ref, bias1_ref,
                  Wt2_ref, as2_ref, ad2_ref, We2_ref, ae2_ref, bias2_ref,
                  A1nf_ref, A1ax_ref, A1ef_ref, c1_ref,
                  A2_ref, c2_ref, A3_ref, c3_ref,
                  out_ref, akl_ref, nkl_ref):
    f32 = jnp.float32
    nf = nf_ref[0]

    h1 = jax.nn.relu(_dot(nf, W1_ref[...]) + b1_ref[...])
    sv = jax.nn.relu(_dot(h1, W2_ref[...]) + b2_ref[...])
    s_col = jnp.tanh(_dot(sv, p_col_ref[...]) / pn_ref[...])
    s_row = _dot_lt(s_col, _eye(NB))

    ii = jax.lax.broadcasted_iota(jnp.int32, (NB, NB), 0)
    jj = jax.lax.broadcasted_iota(jnp.int32, (NB, NB), 1)
    same_g = _iota_col(NB, NG) == _iota_row(NB, NG)
    beats_c = (((s_row > s_col) | ((s_row == s_col) & (jj < ii)))
               & (jj != ii) & same_g)
    beats = jnp.where(beats_c, 1.0, 0.0).astype(f32)
    ones_row = jnp.ones((1, NB), f32)
    cnt_row = _dot_rt(ones_row, beats, _DEF)
    slot_row = cnt_row + (_iota_row(NB, NG) * K).astype(f32)
    p_glob = jax.lax.broadcasted_iota(jnp.int32, (KB, NB), 0).astype(f32)
    sel = jnp.where((p_glob == slot_row) & (cnt_row < float(K)),
                    1.0, 0.0).astype(f32)

    weight = _dot(sel, s_col, _HI)
    af = _dot(sel, sv, _HI) * weight

    ap = _dot(af, Wa_ref[...]) + ba_ref[...]
    npj = _dot(nf, Wn_ref[...]) + bn_ref[...]

    iga = jnp.where(_iota_col(B) == _iota_row(KB, K), 1.0, 0.0).astype(f32)
    mu_a = _dot(iga, ap, _HI) / K
    sq_a = _dot(iga, ap * ap, _HI)
    var_a = (sq_a - K * mu_a * mu_a) / (K - 1)
    akl_ref[0] = 0.5 * jnp.sum(var_a + mu_a * mu_a - 1.0 - jnp.log(var_a),
                               axis=1, keepdims=True)

    ign = jnp.where(_iota_col(B) == _iota_row(NB, NG), 1.0, 0.0).astype(f32)
    mu_n = _dot(ign, npj, _HI) / NG
    sq_n = _dot(ign, npj * npj, _HI)
    var_n = (sq_n - NG * mu_n * mu_n) / (NG - 1)
    nkl_ref[0] = 0.5 * jnp.sum(var_n + mu_n * mu_n - 1.0 - jnp.log(var_n),
                               axis=1, keepdims=True)

    n2 = jnp.sum(npj * npj, axis=1, keepdims=True)
    a2col = jnp.sum(ap * ap, axis=1, keepdims=True)
    a2row = _dot_lt(a2col, _eye(KB))
    cross = _dot_rt(npj, ap, _HI)
    d2 = jnp.maximum(n2 + a2row - 2.0 * cross, 0.0)
    same_a = _iota_col(NB, NG) == _iota_row(KB, K)
    dist = jnp.where(same_a, jnp.sqrt(d2), 1e30)
    dmin = jnp.min(dist, axis=1, keepdims=True)
    kidx = jax.lax.broadcasted_iota(jnp.int32, (NB, KB), 1).astype(f32)
    argm = jnp.min(jnp.where(dist <= dmin, kidx, float(KB)),
                   axis=1, keepdims=True)
    oh = jnp.where(kidx == argm, 1.0, 0.0).astype(f32)
    ohT = _dot_rt(_eye(KB), oh, _HI)

    dscore = 1.0 / jnp.sum(jnp.exp(dmin - dist), axis=1, keepdims=True)
    ef = (npj - _dot(oh, ap, _HI)) * dscore

    aq = _dot(af, Wq_ref[...]) + bq_ref[...]
    kk = _dot(nf, Wk_nf_ref[...]) + _dot(ef, Wk_ef_ref[...]) + bk_ref[...]
    vv = _dot(nf, Wv_nf_ref[...]) + _dot(ef, Wv_ef_ref[...]) + bv_ref[...]
    attn = jnp.sum(_dot(oh, aq, _HI) * kk, axis=1, keepdims=True)

    masked = jnp.where(oh > 0.0, attn, _NEG)
    m_row = jnp.max(masked, axis=0, keepdims=True)
    ex = jnp.exp(attn - jnp.sum(oh * m_row, axis=1, keepdims=True))
    den_row = jnp.sum(oh * ex, axis=0, keepdims=True)
    alpha = ex / (jnp.sum(oh * den_row, axis=1, keepdims=True) + 1e-16)
    upd = _dot(ohT, alpha * vv, _HI)

    gln = gln_ref[...]
    bln = bln_ref[...]

    afu = af + upd
    mu = jnp.mean(afu, axis=1, keepdims=True)
    d = afu - mu
    var = jnp.mean(d * d, axis=1, keepdims=True)
    af2 = d * jax.lax.rsqrt(var + 1e-5) * gln + bln

    same_aa = _iota_col(KB, K) == _iota_row(KB, K)

    def gat(x, Wt_ref, as_ref, ad_ref, We_ref, ae_ref, b_ref):
        xt = _dot(x, Wt_ref[...])
        ddot = _dot(xt, ad_ref[...])
        sdot = _dot_rt(as_ref[...], xt)
        we = _dot(We_ref[...], ae_ref[...], _HI)
        ce_col = _dot(ap, we, _HI)
        ce_row = _dot_lt(ce_col, _eye(KB))
        lg = (ddot - ce_col) + (sdot + ce_row)
        lg = jnp.where(lg >= 0.0, lg, 0.2 * lg)
        lg = jnp.where(same_aa, lg, _NEG)
        rmax = jnp.max(lg, axis=1, keepdims=True)
        e = jnp.exp(lg - rmax)
        a = e / (jnp.sum(e, axis=1, keepdims=True) + 1e-16)
        return _dot(a, xt) + b_ref[...]

    h = jax.nn.relu(gat(af2, Wt1_ref, as1_ref, ad1_ref, We1_ref,
                        ae1_ref, bias1_ref))
    af3 = gat(h, Wt2_ref, as2_ref, ad2_ref, We2_ref,
              ae2_ref, bias2_ref)

    ax = _dot(oh, af3, _HI)
    m1 = jax.nn.relu(_dot(nf, A1nf_ref[...]) + _dot(ax, A1ax_ref[...])
                     + _dot(-ef, A1ef_ref[...]) + c1_ref[...])
    m2 = jax.nn.relu(_dot(m1, A2_ref[...]) + c2_ref[...])
    x = nf + _dot(m2, A3_ref[...]) + c3_ref[...]
    mu = jnp.mean(x, axis=1, keepdims=True)
    d = x - mu
    var = jnp.mean(d * d, axis=1, keepdims=True)
    out_ref[0] = d * jax.lax.rsqrt(var + 1e-5) * gln + bln


def _full(shape):
    nd = len(shape)
    return pl.BlockSpec(shape, lambda g, _nd=nd: (0,) * _nd)


@jax.jit
def kernel(node_x, node_features, edge_index, batch, W1, b1, W2, b2, p,
           Wa, ba, Wn, bn, Wq, bq, Wkv, bkv, g_ln, b_ln,
           Wt1, as1, ad1, We1, ae1, bias1, Wt2, as2, ad2, We2, ae2, bias2,
           A1, c1, A2, c2, A3, c3):
    f32 = jnp.float32
    nf3 = node_features.reshape(G // B, NB, H)
    row = lambda v: v.reshape(1, -1).astype(f32)
    col = lambda v: v.reshape(-1, 1).astype(f32)
    pn = jnp.linalg.norm(p).reshape(1, 1)

    operands = [
        nf3, col(p), pn,
        W1, row(b1), W2, row(b2),
        Wa, row(ba), Wn, row(bn),
        Wq, row(bq),
        Wkv[:H, :H], Wkv[H:, :H], row(bkv[:H]),
        Wkv[:H, H:], Wkv[H:, H:], row(bkv[H:]),
        row(g_ln), row(b_ln),
        Wt1, row(as1), col(ad1), We1, col(ae1), row(bias1),
        Wt2, row(as2), col(ad2), We2, col(ae2), row(bias2),
        A1[:H], A1[H:2 * H], A1[2 * H:], row(c1),
        A2, row(c2), A3, row(c3),
    ]
    in_specs = [pl.BlockSpec((1, NB, H), lambda g: (g, 0, 0))]
    in_specs += [_full(op.shape) for op in operands[1:]]

    out_shapes = (
        jax.ShapeDtypeStruct((G // B, NB, H), f32),
        jax.ShapeDtypeStruct((G // B, B, 1), f32),
        jax.ShapeDtypeStruct((G // B, B, 1), f32),
    )
    out_specs = (
        pl.BlockSpec((1, NB, H), lambda g: (g, 0, 0)),
        pl.BlockSpec((1, B, 1), lambda g: (g, 0, 0)),
        pl.BlockSpec((1, B, 1), lambda g: (g, 0, 0)),
    )

    node_out, akl, nkl = pl.pallas_call(
        _graph_kernel,
        grid=(G // B,),
        in_specs=in_specs,
        out_specs=out_specs,
        out_shape=out_shapes,
        compiler_params=pltpu.CompilerParams(
            dimension_semantics=("arbitrary",)),
    )(*operands)

    return node_out.reshape(N, H), akl.reshape(G), nkl.reshape(G)


# speedup vs baseline: 69.2337x; 1.2926x over previous
import jax
import jax.numpy as jnp
from jax.experimental import pallas as pl
from jax.experimental.pallas import tpu as pltpu

G = 50
NG = 200
N = G * NG
H = 128
CD = 16
K = 50

B = 5
NB = B * NG
KB = B * K

_DEF = jax.lax.Precision.DEFAULT
_HI = jax.lax.Precision.HIGHEST
_NEG = -1e30


def _dot(a, b, prec=_DEF):
    return jax.lax.dot_general(a, b, (((1,), (0,)), ((), ())),
                               precision=prec,
                               preferred_element_type=jnp.float32)


def _dot_rt(a, b, prec=_DEF):
    return jax.lax.dot_general(a, b, (((1,), (1,)), ((), ())),
                               precision=prec,
                               preferred_element_type=jnp.float32)


def _dot_lt(a, b, prec=_HI):
    return jax.lax.dot_general(a, b, (((0,), (0,)), ((), ())),
                               precision=prec,
                               preferred_element_type=jnp.float32)


def _split3(v):
    hi = v.astype(jnp.bfloat16).astype(jnp.float32)
    r = v - hi
    mid = r.astype(jnp.bfloat16).astype(jnp.float32)
    return hi, mid, r - mid


def _gather(onehot, vals):
    hi, mid, lo = _split3(vals)
    return (_dot(onehot, hi) + _dot(onehot, mid)) + _dot(onehot, lo)


def _transpose(col, eye):
    hi, mid, lo = _split3(col)
    return (_dot_lt(hi, eye, _DEF) + _dot_lt(mid, eye, _DEF)) + \
        _dot_lt(lo, eye, _DEF)


def _eye(n):
    return jnp.where(jax.lax.broadcasted_iota(jnp.int32, (n, n), 0) ==
                     jax.lax.broadcasted_iota(jnp.int32, (n, n), 1),
                     1.0, 0.0).astype(jnp.float32)


def _iota_col(n, div=1):
    return jax.lax.broadcasted_iota(jnp.int32, (n, 1), 0) // div


def _iota_row(n, div=1):
    return jax.lax.broadcasted_iota(jnp.int32, (1, n), 1) // div


def _graph_kernel(nf_ref, p_col_ref, pn_ref,
                  W1_ref, b1_ref, W2_ref, b2_ref,
                  Wa_ref, ba_ref, Wn_ref, bn_ref,
                  Wq_ref, bq_ref,
                  Wkv_nf_ref, Wkv_ef_ref, bk_ref, bv_ref,
                  gln_ref, bln_ref,
                  Wt1_ref, as1_ref, ad1_ref, We1_ref, ae1_ref, bias1_ref,
                  Wt2_ref, as2_ref, ad2_ref, We2_ref, ae2_ref, bias2_ref,
                  A1nf_ref, A1ax_ref, A1ef_ref, c1_ref,
                  A2_ref, c2_ref, A3_ref, c3_ref,
                  out_ref, akl_ref, nkl_ref):
    f32 = jnp.float32
    nf = nf_ref[0]

    h1 = jax.nn.relu(_dot(nf, W1_ref[...]) + b1_ref[...])
    sv = jax.nn.relu(_dot(h1, W2_ref[...]) + b2_ref[...])
    s_col = jnp.tanh(_dot(sv, p_col_ref[...]) / pn_ref[...])
    s_row = _transpose(s_col, _eye(NB))

    ii = jax.lax.broadcasted_iota(jnp.int32, (NB, NB), 0)
    jj = jax.lax.broadcasted_iota(jnp.int32, (NB, NB), 1)
    same_g = _iota_col(NB, NG) == _iota_row(NB, NG)
    beats_c = (((s_row > s_col) | ((s_row == s_col) & (jj < ii)))
               & (jj != ii) & same_g)
    beats = jnp.where(beats_c, 1.0, 0.0).astype(f32)
    ones_row = jnp.ones((1, NB), f32)
    cnt_row = _dot_rt(ones_row, beats, _DEF)
    slot_row = cnt_row + (_iota_row(NB, NG) * K).astype(f32)
    p_glob = jax.lax.broadcasted_iota(jnp.int32, (KB, NB), 0).astype(f32)
    sel = jnp.where((p_glob == slot_row) & (cnt_row < float(K)),
                    1.0, 0.0).astype(f32)

    weight = _gather(sel, s_col)
    af = _gather(sel, sv) * weight

    ap = _dot(af, Wa_ref[...]) + ba_ref[...]
    npj = _dot(nf, Wn_ref[...]) + bn_ref[...]

    iga = jnp.where(_iota_col(B) == _iota_row(KB, K), 1.0, 0.0).astype(f32)
    mu_a = _dot(iga, ap, _HI) / K
    sq_a = _dot(iga, ap * ap, _HI)
    var_a = (sq_a - K * mu_a * mu_a) / (K - 1)
    akl_ref[0] = 0.5 * jnp.sum(var_a + mu_a * mu_a - 1.0 - jnp.log(var_a),
                               axis=1, keepdims=True)

    ign = jnp.where(_iota_col(B) == _iota_row(NB, NG), 1.0, 0.0).astype(f32)
    mu_n = _dot(ign, npj, _HI) / NG
    sq_n = _dot(ign, npj * npj, _HI)
    var_n = (sq_n - NG * mu_n * mu_n) / (NG - 1)
    nkl_ref[0] = 0.5 * jnp.sum(var_n + mu_n * mu_n - 1.0 - jnp.log(var_n),
                               axis=1, keepdims=True)

    eyeK = _eye(KB)
    n2 = jnp.sum(npj * npj, axis=1, keepdims=True)
    a2col = jnp.sum(ap * ap, axis=1, keepdims=True)
    a2row = _transpose(a2col, eyeK)
    cross = _dot_rt(npj, ap, _HI)
    d2 = jnp.maximum(n2 + a2row - 2.0 * cross, 0.0)
    same_a = _iota_col(NB, NG) == _iota_row(KB, K)
    dist = jnp.where(same_a, jnp.sqrt(d2), 1e30)
    dmin = jnp.min(dist, axis=1, keepdims=True)
    kidx = jax.lax.broadcasted_iota(jnp.int32, (NB, KB), 1).astype(f32)
    argm = jnp.min(jnp.where(dist <= dmin, kidx, float(KB)),
                   axis=1, keepdims=True)
    oh = jnp.where(kidx == argm, 1.0, 0.0).astype(f32)
    ohT = _dot_rt(eyeK, oh, _DEF)

    dscore = 1.0 / jnp.sum(jnp.exp(dmin - dist), axis=1, keepdims=True)
    ef = (npj - _gather(oh, ap)) * dscore

    aq = _dot(af, Wq_ref[...]) + bq_ref[...]
    nf_kv = _dot(nf, Wkv_nf_ref[...])
    ef_kv = _dot(ef, Wkv_ef_ref[...])
    kk = nf_kv[:, :H] + ef_kv[:, :H] + bk_ref[...]
    vv = nf_kv[:, H:] + ef_kv[:, H:] + bv_ref[...]
    attn = jnp.sum(_gather(oh, aq) * kk, axis=1, keepdims=True)

    masked = jnp.where(oh > 0.0, attn, _NEG)
    m_row = jnp.max(masked, axis=0, keepdims=True)
    ex = jnp.exp(attn - jnp.sum(oh * m_row, axis=1, keepdims=True))
    den_row = jnp.sum(oh * ex, axis=0, keepdims=True)
    alpha = ex / (jnp.sum(oh * den_row, axis=1, keepdims=True) + 1e-16)
    upd = _gather(ohT, alpha * vv)

    gln = gln_ref[...]
    bln = bln_ref[...]

    afu = af + upd
    mu = jnp.mean(afu, axis=1, keepdims=True)
    d = afu - mu
    var = jnp.mean(d * d, axis=1, keepdims=True)
    af2 = d * jax.lax.rsqrt(var + 1e-5) * gln + bln

    same_aa = _iota_col(KB, K) == _iota_row(KB, K)

    def gat(x, Wt_ref, as_ref, ad_ref, We_ref, ae_ref, b_ref):
        xt = _dot(x, Wt_ref[...])
        ddot = _dot(xt, ad_ref[...])
        sdot = _dot_rt(as_ref[...], xt)
        we = _dot(We_ref[...], ae_ref[...], _HI)
        ce_col = _dot(ap, we, _HI)
        ce_row = _transpose(ce_col, eyeK)
        lg = (ddot - ce_col) + (sdot + ce_row)
        lg = jnp.where(lg >= 0.0, lg, 0.2 * lg)
        lg = jnp.where(same_aa, lg, _NEG)
        rmax = jnp.max(lg, axis=1, keepdims=True)
        e = jnp.exp(lg - rmax)
        a = e / (jnp.sum(e, axis=1, keepdims=True) + 1e-16)
        return _dot(a, xt) + b_ref[...]

    h = jax.nn.relu(gat(af2, Wt1_ref, as1_ref, ad1_ref, We1_ref,
                        ae1_ref, bias1_ref))
    af3 = gat(h, Wt2_ref, as2_ref, ad2_ref, We2_ref,
              ae2_ref, bias2_ref)

    ax = _gather(oh, af3)
    m1 = jax.nn.relu(_dot(nf, A1nf_ref[...]) + _dot(ax, A1ax_ref[...])
                     + _dot(-ef, A1ef_ref[...]) + c1_ref[...])
    m2 = jax.nn.relu(_dot(m1, A2_ref[...]) + c2_ref[...])
    x = nf + _dot(m2, A3_ref[...]) + c3_ref[...]
    mu = jnp.mean(x, axis=1, keepdims=True)
    d = x - mu
    var = jnp.mean(d * d, axis=1, keepdims=True)
    out_ref[0] = d * jax.lax.rsqrt(var + 1e-5) * gln + bln


def _full(shape):
    nd = len(shape)
    return pl.BlockSpec(shape, lambda g, _nd=nd: (0,) * _nd)


@jax.jit
def kernel(node_x, node_features, edge_index, batch, W1, b1, W2, b2, p,
           Wa, ba, Wn, bn, Wq, bq, Wkv, bkv, g_ln, b_ln,
           Wt1, as1, ad1, We1, ae1, bias1, Wt2, as2, ad2, We2, ae2, bias2,
           A1, c1, A2, c2, A3, c3):
    f32 = jnp.float32
    nf3 = node_features.reshape(G // B, NB, H)
    row = lambda v: v.reshape(1, -1).astype(f32)
    col = lambda v: v.reshape(-1, 1).astype(f32)
    pn = jnp.linalg.norm(p).reshape(1, 1)

    operands = [
        nf3, col(p), pn,
        W1, row(b1), W2, row(b2),
        Wa, row(ba), Wn, row(bn),
        Wq, row(bq),
        Wkv[:H, :], Wkv[H:, :], row(bkv[:H]), row(bkv[H:]),
        row(g_ln), row(b_ln),
        Wt1, row(as1), col(ad1), We1, col(ae1), row(bias1),
        Wt2, row(as2), col(ad2), We2, col(ae2), row(bias2),
        A1[:H], A1[H:2 * H], A1[2 * H:], row(c1),
        A2, row(c2), A3, row(c3),
    ]
    in_specs = [pl.BlockSpec((1, NB, H), lambda g: (g, 0, 0))]
    in_specs += [_full(op.shape) for op in operands[1:]]

    out_shapes = (
        jax.ShapeDtypeStruct((G // B, NB, H), f32),
        jax.ShapeDtypeStruct((G // B, B, 1), f32),
        jax.ShapeDtypeStruct((G // B, B, 1), f32),
    )
    out_specs = (
        pl.BlockSpec((1, NB, H), lambda g: (g, 0, 0)),
        pl.BlockSpec((1, B, 1), lambda g: (g, 0, 0)),
        pl.BlockSpec((1, B, 1), lambda g: (g, 0, 0)),
    )

    node_out, akl, nkl = pl.pallas_call(
        _graph_kernel,
        grid=(G // B,),
        in_specs=in_specs,
        out_specs=out_specs,
        out_shape=out_shapes,
        compiler_params=pltpu.CompilerParams(
            dimension_semantics=("arbitrary",)),
    )(*operands)

    return node_out.reshape(N, H), akl.reshape(G), nkl.reshape(G)
